# Initial kernel scaffold; baseline (speedup 1.0000x reference)
import jax
import jax.numpy as jnp
from jax.experimental import pallas as pl


def kernel(xyz, t, data):
    raise NotImplementedError("write your pallas kernel here")


# speedup vs baseline: 1.4974x; 1.4974x over previous
import jax
import jax.numpy as jnp
import numpy as np
from jax import lax
from jax.experimental import pallas as pl
from jax.experimental.pallas import tpu as pltpu
from jax.experimental.pallas import tpu_sc as plsc

N_PTS = 32768
NL = 16
F = 2


def _consts():
    nl = NL
    b = 1.38
    base = 16
    P = 2 ** 19
    def isprime(n):
        i = 2
        while i * i <= n:
            if n % i == 0:
                return False
            i += 1
        return True
    while not isprime(P):
        P += 1
    xyz_min = np.array([-1.0, -1.0, -1.0])
    xyz_max = np.array([1.0, 1.0, 1.0])
    ens, enn = [], []
    for i in range(nl):
        grid_num = int((base * b ** i) ** 3)
        grid_size = ((xyz_max - xyz_min).prod() / grid_num) ** (1.0 / 3.0)
        world_size = (xyz_max - xyz_min) / grid_size
        xyzt_num = np.concatenate([world_size, np.array([100])]).astype(np.int32)
        enn.append(xyzt_num)
        xyz_size = (xyz_max - xyz_min) / (world_size - 1)
        xyzt_size = np.concatenate([xyz_size, np.array([1.0 / 99])])
        ens.append(xyzt_size)
    return P, np.array(ens, dtype=np.float32), np.array(enn, dtype=np.int32)


_P, _ENS, _ENN = _consts()
_R = _P - (1 << 19)
_PS = (1, 19349663, 83492791, 73856093)

NW = 32
NP = N_PTS // NW
NPH = NP // 2
NGH = NPH // 16
CH = 128
NCHH = 16 * NPH // CH


def _mod_p(v):
    a = lax.shift_right_logical(v, np.int32(19))
    c = jnp.bitwise_and(v, np.int32((1 << 19) - 1))
    m = c - _R * a
    return jnp.where(m < 0, m + _P, m)


def _sc_body(coords, fcoh, icoh, datah, outh,
             cvm, fcv, icv, idxv, rows, wbuf, obuf, sem):
    wid = lax.axis_index("s") * 2 + lax.axis_index("c")
    base = wid * NP
    pltpu.sync_copy(coords.at[:, pl.ds(base, NP)], cvm)
    pltpu.sync_copy(fcoh, fcv)
    pltpu.sync_copy(icoh, icv)
    iota = lax.broadcasted_iota(jnp.int32, (16,), 0)
    zeros = jnp.zeros((16,), jnp.int32)
    ones = jnp.ones((16,), jnp.int32)

    def index_pass(l, lbase, is_direct, p0):
        fvec = fcv[pl.ds(4 * l, 16)]
        ivec = icv[pl.ds(4 * l, 16)]
        es = [fvec[d] for d in range(4)]
        nm1 = [ivec[d] for d in range(4)]

        def grp(_, g):
            col = 16 * (g & np.int32(7))
            row8 = g >> np.int32(3)
            hi8 = []
            lo8 = []
            w01 = []
            for d in range(4):
                x = cvm[d, pl.ds(p0 + 16 * g, 16)]
                if d < 3:
                    fx = (x + np.float32(1.0)) / es[d]
                else:
                    fx = x / es[d]
                i0 = fx.astype(jnp.int32)
                i1 = (fx + np.float32(1.0)).astype(jnp.int32)
                frac = fx - i0.astype(jnp.float32)
                i0 = jnp.maximum(jnp.minimum(i0, nm1[d]), 0)
                i1 = jnp.maximum(jnp.minimum(i1, nm1[d]), 0)
                w01.append((np.float32(1.0) - frac, frac))
                if is_direct:
                    hi8.append((i0, i1))
                else:
                    chi = _PS[d] >> 8
                    clo = _PS[d] & 255
                    hs = []
                    ls = []
                    for v in (i0, i1):
                        tlo = v * clo
                        ls.append(jnp.bitwise_and(tlo, np.int32(255)))
                        hs.append(v * chi
                                  + lax.shift_right_logical(tlo, np.int32(8)))
                    hi8.append(tuple(hs))
                    lo8.append(tuple(ls))
            for k in range(16):
                bits = ((k >> 3) & 1, (k >> 2) & 1, (k >> 1) & 1, k & 1)
                if is_direct:
                    e1 = int(_ENN[0][1]); e2 = int(_ENN[0][2]); e3 = int(_ENN[0][3])
                    ind = (hi8[0][bits[0]] * (e1 * e2 * e3)
                           + hi8[1][bits[1]] * (e2 * e3)
                           + hi8[2][bits[2]] * e3
                           + hi8[3][bits[3]])
                else:
                    xh = (hi8[0][bits[0]] ^ hi8[1][bits[1]]
                          ^ hi8[2][bits[2]] ^ hi8[3][bits[3]])
                    xl = (lo8[0][bits[0]] ^ lo8[1][bits[1]]
                          ^ lo8[2][bits[2]] ^ lo8[3][bits[3]])
                    ind = _mod_p(_mod_p(xh) * 256 + xl)
                w = w01[0][bits[0]] * w01[1][bits[1]]
                w = w * w01[2][bits[2]]
                w = w * w01[3][bits[3]]
                idxv[k * (NPH // CH) + row8, pl.ds(col, 16)] = ind + lbase
                wbuf[k, pl.ds(16 * g, 16)] = w
            return g + 1

        lax.fori_loop(0, NGH, grp, np.int32(0))

    def fire(_, j):
        pltpu.async_copy(datah.at[idxv.at[j]], rows.at[pl.ds(j * CH, CH)], sem)
        return j + 1

    def drain(_, j):
        pltpu.make_async_copy(
            datah.at[idxv.at[j]], rows.at[pl.ds(j * CH, CH)], sem).wait()
        return j + 1

    def combine_pass(l, p0):
        def grp(_, g):
            n16 = 16 * g + iota
            acc0 = jnp.zeros((16,), jnp.float32)
            acc1 = jnp.zeros((16,), jnp.float32)
            for k in range(16):
                r = k * NPH + n16
                f0 = plsc.load_gather(rows, [r, zeros])
                f1 = plsc.load_gather(rows, [r, ones])
                w = wbuf[k, pl.ds(16 * g, 16)]
                acc0 = acc0 + w * f0
                acc1 = acc1 + w * f1
            c0 = jnp.full((16,), 2 * l, jnp.int32)
            plsc.store_scatter(obuf, [p0 + n16, c0], acc0)
            plsc.store_scatter(obuf, [p0 + n16, c0 + 1], acc1)
            return g + 1

        lax.fori_loop(0, NGH, grp, np.int32(0))

    def run_level(l, lbase, is_direct):
        for h in range(2):
            p0 = h * NPH
            index_pass(l, lbase, is_direct, p0)
            lax.fori_loop(0, NCHH, fire, np.int32(0))
            lax.fori_loop(0, NCHH, drain, np.int32(0))
            combine_pass(l, p0)

    run_level(0, 0, True)

    def hashed(_, l):
        run_level(l, l * _P, False)
        return l + 1

    lax.fori_loop(1, NL, hashed, np.int32(1))
    pltpu.sync_copy(obuf, outh.at[pl.ds(base, NP)])


@jax.jit
def _run(coords, fco, ico, dataf):
    mesh = plsc.VectorSubcoreMesh(core_axis_name="c", subcore_axis_name="s")
    return pl.kernel(
        _sc_body,
        out_type=jax.ShapeDtypeStruct((N_PTS, NL * F), jnp.float32),
        mesh=mesh,
        compiler_params=pltpu.CompilerParams(
            needs_layout_passes=False, use_tc_tiling_on_sc=False),
        scratch_types=[
            pltpu.VMEM((4, NP), jnp.float32),
            pltpu.VMEM((4 * NL + 16,), jnp.float32),
            pltpu.VMEM((4 * NL + 16,), jnp.int32),
            pltpu.VMEM((NCHH, CH), jnp.int32),
            pltpu.VMEM((NCHH * CH, F), jnp.float32),
            pltpu.VMEM((16, NPH), jnp.float32),
            pltpu.VMEM((NP, NL * F), jnp.float32),
            pltpu.SemaphoreType.DMA,
        ],
    )(coords, fco, ico, dataf)


def kernel(xyz, t, data):
    coords = jnp.concatenate([xyz, t], axis=1).astype(jnp.float32).T
    fco = jnp.asarray(np.pad(_ENS.reshape(-1), (0, 16)))
    ico = jnp.asarray(np.pad(_ENN.reshape(-1) - 1, (0, 16)))
    dataf = data.astype(jnp.float32).reshape(NL * _P, F)
    return _run(coords, fco, ico, dataf)


# speedup vs baseline: 9.0852x; 6.0672x over previous
import jax
import jax.numpy as jnp
import numpy as np
from jax import lax
from jax.experimental import pallas as pl
from jax.experimental.pallas import tpu as pltpu
from jax.experimental.pallas import tpu_sc as plsc

N_PTS = 32768
NL = 16
F = 2


def _consts():
    nl = NL
    b = 1.38
    base = 16
    P = 2 ** 19
    def isprime(n):
        i = 2
        while i * i <= n:
            if n % i == 0:
                return False
            i += 1
        return True
    while not isprime(P):
        P += 1
    xyz_min = np.array([-1.0, -1.0, -1.0])
    xyz_max = np.array([1.0, 1.0, 1.0])
    ens, enn = [], []
    for i in range(nl):
        grid_num = int((base * b ** i) ** 3)
        grid_size = ((xyz_max - xyz_min).prod() / grid_num) ** (1.0 / 3.0)
        world_size = (xyz_max - xyz_min) / grid_size
        xyzt_num = np.concatenate([world_size, np.array([100])]).astype(np.int32)
        enn.append(xyzt_num)
        xyz_size = (xyz_max - xyz_min) / (world_size - 1)
        xyzt_size = np.concatenate([xyz_size, np.array([1.0 / 99])])
        ens.append(xyzt_size)
    return P, np.array(ens, dtype=np.float32), np.array(enn, dtype=np.int32)


_P, _ENS, _ENN = _consts()
_R = _P - (1 << 19)
_PS = (1, 19349663, 83492791, 73856093)

NW = 32
NP = N_PTS // NW
NPH = NP // 2
NGH = NPH // 16
CH = 128
NCH2 = 2 * 16 * NPH // CH
FOFF = 16 * NPH


def _mod_p(v):
    a = lax.shift_right_logical(v, np.int32(19))
    c = jnp.bitwise_and(v, np.int32((1 << 19) - 1))
    m = c - _R * a
    return jnp.where(m < 0, m + _P, m)


def _sc_body(coords, fcoh, icoh, datah, outh,
             cvm, fcv, icv, idxv, rows, wbuf, obuf, tbl, sem):
    sid = lax.axis_index("s")
    wid = sid * 2 + lax.axis_index("c")
    base = wid * NP
    pltpu.sync_copy(coords.at[:, pl.ds(base, NP)], cvm)
    pltpu.sync_copy(fcoh, fcv)
    pltpu.sync_copy(icoh, icv)
    iota = lax.broadcasted_iota(jnp.int32, (16,), 0)

    def index_pass(l, is_direct, p0):
        fvec = fcv[pl.ds(4 * l, 16)]
        ivec = icv[pl.ds(4 * l, 16)]
        es = [fvec[d] for d in range(4)]
        nm1 = [ivec[d] for d in range(4)]

        def grp(_, g):
            col = 16 * (g & np.int32(7))
            row8 = g >> np.int32(3)
            hi8 = []
            lo8 = []
            w01 = []
            for d in range(4):
                x = cvm[d, pl.ds(p0 + 16 * g, 16)]
                if d < 3:
                    fx = (x + np.float32(1.0)) / es[d]
                else:
                    fx = x / es[d]
                i0 = fx.astype(jnp.int32)
                i1 = (fx + np.float32(1.0)).astype(jnp.int32)
                frac = fx - i0.astype(jnp.float32)
                i0 = jnp.maximum(jnp.minimum(i0, nm1[d]), 0)
                i1 = jnp.maximum(jnp.minimum(i1, nm1[d]), 0)
                w01.append((np.float32(1.0) - frac, frac))
                if is_direct:
                    hi8.append((i0, i1))
                else:
                    chi = _PS[d] >> 8
                    clo = _PS[d] & 255
                    hs = []
                    ls = []
                    for v in (i0, i1):
                        tlo = v * clo
                        ls.append(jnp.bitwise_and(tlo, np.int32(255)))
                        hs.append(v * chi
                                  + lax.shift_right_logical(tlo, np.int32(8)))
                    hi8.append(tuple(hs))
                    lo8.append(tuple(ls))
            for k in range(16):
                bits = ((k >> 3) & 1, (k >> 2) & 1, (k >> 1) & 1, k & 1)
                if is_direct:
                    e1 = int(_ENN[0][1]); e2 = int(_ENN[0][2]); e3 = int(_ENN[0][3])
                    ind = (hi8[0][bits[0]] * (e1 * e2 * e3)
                           + hi8[1][bits[1]] * (e2 * e3)
                           + hi8[2][bits[2]] * e3
                           + hi8[3][bits[3]])
                else:
                    xh = (hi8[0][bits[0]] ^ hi8[1][bits[1]]
                          ^ hi8[2][bits[2]] ^ hi8[3][bits[3]])
                    xl = (lo8[0][bits[0]] ^ lo8[1][bits[1]]
                          ^ lo8[2][bits[2]] ^ lo8[3][bits[3]])
                    ind = _mod_p(_mod_p(xh) * 256 + xl)
                w = w01[0][bits[0]] * w01[1][bits[1]]
                w = w * w01[2][bits[2]]
                w = w * w01[3][bits[3]]
                ind2 = ind + ind
                idxv[k * (NPH // CH) + row8, pl.ds(col, 16)] = ind2
                idxv[(FOFF // CH) + k * (NPH // CH) + row8,
                     pl.ds(col, 16)] = ind2 + 1
                wbuf[k, pl.ds(16 * g, 16)] = w
            return g + 1

        lax.fori_loop(0, NGH, grp, np.int32(0))

    def stage(l):
        @pl.when(sid == 0)
        def _():
            pltpu.sync_copy(datah.at[l], tbl)
        plsc.subcore_barrier()

    def fire(_, j):
        pltpu.async_copy(tbl.at[idxv.at[j]], rows.at[pl.ds(j * CH, CH)], sem)
        return j + 1

    def drain(_, j):
        pltpu.make_async_copy(
            tbl.at[idxv.at[j]], rows.at[pl.ds(j * CH, CH)], sem).wait()
        return j + 1

    def combine_pass(l):
        def grp(_, g):
            n16 = 16 * g + iota
            acc0 = jnp.zeros((16,), jnp.float32)
            acc1 = jnp.zeros((16,), jnp.float32)
            for k in range(16):
                r = k * NPH + n16
                f0 = plsc.load_gather(rows, [r])
                f1 = plsc.load_gather(rows, [FOFF + r])
                w = wbuf[k, pl.ds(16 * g, 16)]
                acc0 = acc0 + w * f0
                acc1 = acc1 + w * f1
            c0 = jnp.full((16,), 2 * l, jnp.int32)
            plsc.store_scatter(obuf, [n16, c0], acc0)
            plsc.store_scatter(obuf, [n16, c0 + 1], acc1)
            return g + 1

        lax.fori_loop(0, NGH, grp, np.int32(0))

    for h in range(2):
        p0 = h * NPH

        def run_level(l, is_direct, p0=p0):
            stage(l)
            index_pass(l, is_direct, p0)
            lax.fori_loop(0, NCH2, fire, np.int32(0))
            lax.fori_loop(0, NCH2, drain, np.int32(0))
            combine_pass(l)
            plsc.subcore_barrier()

        run_level(jnp.int32(0), True)

        def hashed(_, l, run_level=run_level):
            run_level(l, False)
            return l + 1

        lax.fori_loop(1, NL, hashed, np.int32(1))
        pltpu.sync_copy(obuf, outh.at[pl.ds(base + p0, NPH)])


@jax.jit
def _run(coords, fco, ico, dataf):
    mesh = plsc.VectorSubcoreMesh(core_axis_name="c", subcore_axis_name="s")
    return pl.kernel(
        _sc_body,
        out_type=jax.ShapeDtypeStruct((N_PTS, NL * F), jnp.float32),
        mesh=mesh,
        compiler_params=pltpu.CompilerParams(
            needs_layout_passes=False, use_tc_tiling_on_sc=False),
        scratch_types=[
            pltpu.VMEM((4, NP), jnp.float32),
            pltpu.VMEM((4 * NL + 16,), jnp.float32),
            pltpu.VMEM((4 * NL + 16,), jnp.int32),
            pltpu.VMEM((NCH2, CH), jnp.int32),
            pltpu.VMEM((2 * 16 * NPH,), jnp.float32),
            pltpu.VMEM((16, NPH), jnp.float32),
            pltpu.VMEM((NPH, NL * F), jnp.float32),
            pltpu.VMEM_SHARED((_P * F,), jnp.float32),
            pltpu.SemaphoreType.DMA,
        ],
    )(coords, fco, ico, dataf)


def kernel(xyz, t, data):
    coords = jnp.concatenate([xyz, t], axis=1).astype(jnp.float32).T
    fco = jnp.asarray(np.pad(_ENS.reshape(-1), (0, 16)))
    ico = jnp.asarray(np.pad(_ENN.reshape(-1) - 1, (0, 16)))
    dataf = data.astype(jnp.float32).reshape(NL, _P * F)
    return _run(coords, fco, ico, dataf)


# speedup vs baseline: 9.4902x; 1.0446x over previous
import jax
import jax.numpy as jnp
import numpy as np
from jax import lax
from jax.experimental import pallas as pl
from jax.experimental.pallas import tpu as pltpu
from jax.experimental.pallas import tpu_sc as plsc

N_PTS = 32768
NL = 16
F = 2


def _consts():
    nl = NL
    b = 1.38
    base = 16
    P = 2 ** 19
    def isprime(n):
        i = 2
        while i * i <= n:
            if n % i == 0:
                return False
            i += 1
        return True
    while not isprime(P):
        P += 1
    xyz_min = np.array([-1.0, -1.0, -1.0])
    xyz_max = np.array([1.0, 1.0, 1.0])
    ens, enn = [], []
    for i in range(nl):
        grid_num = int((base * b ** i) ** 3)
        grid_size = ((xyz_max - xyz_min).prod() / grid_num) ** (1.0 / 3.0)
        world_size = (xyz_max - xyz_min) / grid_size
        xyzt_num = np.concatenate([world_size, np.array([100])]).astype(np.int32)
        enn.append(xyzt_num)
        xyz_size = (xyz_max - xyz_min) / (world_size - 1)
        xyzt_size = np.concatenate([xyz_size, np.array([1.0 / 99])])
        ens.append(xyzt_size)
    return P, np.array(ens, dtype=np.float32), np.array(enn, dtype=np.int32)


_P, _ENS, _ENN = _consts()
_R = _P - (1 << 19)
_PS = (1, 19349663, 83492791, 73856093)

NW = 32
NP = N_PTS // NW
NPH = NP // 2
NGH = NPH // 16
CH = 128
NCH2 = 2 * 16 * NPH // CH
FOFF = 16 * NPH


def _mod_p(v):
    a = lax.shift_right_logical(v, np.int32(19))
    c = jnp.bitwise_and(v, np.int32((1 << 19) - 1))
    m = c - _R * a
    return jnp.where(m < 0, m + _P, m)


def _sc_body(coords, fcoh, icoh, datah, outh,
             cvm, fcv, icv, idxv, rows, wbuf, obuf, tbl, sem, sem2):
    sid = lax.axis_index("s")
    wid = sid * 2 + lax.axis_index("c")
    base = wid * NP
    pltpu.sync_copy(coords.at[:, pl.ds(base, NP)], cvm)
    pltpu.sync_copy(fcoh, fcv)
    pltpu.sync_copy(icoh, icv)
    iota = lax.broadcasted_iota(jnp.int32, (16,), 0)

    def make_grp(l, is_direct, p0):
        fvec = fcv[pl.ds(4 * l, 16)]
        ivec = icv[pl.ds(4 * l, 16)]
        es = [fvec[d] for d in range(4)]
        nm1 = [ivec[d] for d in range(4)]

        def grp(_, g):
            col = 16 * (g & np.int32(7))
            row8 = g >> np.int32(3)
            hi8 = []
            lo8 = []
            w01 = []
            for d in range(4):
                x = cvm[d, pl.ds(p0 + 16 * g, 16)]
                if d < 3:
                    fx = (x + np.float32(1.0)) / es[d]
                else:
                    fx = x / es[d]
                i0 = fx.astype(jnp.int32)
                i1 = (fx + np.float32(1.0)).astype(jnp.int32)
                frac = fx - i0.astype(jnp.float32)
                i0 = jnp.maximum(jnp.minimum(i0, nm1[d]), 0)
                i1 = jnp.maximum(jnp.minimum(i1, nm1[d]), 0)
                w01.append((np.float32(1.0) - frac, frac))
                if is_direct:
                    hi8.append((i0, i1))
                else:
                    chi = _PS[d] >> 8
                    clo = _PS[d] & 255
                    hs = []
                    ls = []
                    for v in (i0, i1):
                        tlo = v * clo
                        ls.append(jnp.bitwise_and(tlo, np.int32(255)))
                        hs.append(v * chi
                                  + lax.shift_right_logical(tlo, np.int32(8)))
                    hi8.append(tuple(hs))
                    lo8.append(tuple(ls))
            for k in range(16):
                bits = ((k >> 3) & 1, (k >> 2) & 1, (k >> 1) & 1, k & 1)
                if is_direct:
                    e1 = int(_ENN[0][1]); e2 = int(_ENN[0][2]); e3 = int(_ENN[0][3])
                    ind = (hi8[0][bits[0]] * (e1 * e2 * e3)
                           + hi8[1][bits[1]] * (e2 * e3)
                           + hi8[2][bits[2]] * e3
                           + hi8[3][bits[3]])
                else:
                    xh = (hi8[0][bits[0]] ^ hi8[1][bits[1]]
                          ^ hi8[2][bits[2]] ^ hi8[3][bits[3]])
                    xl = (lo8[0][bits[0]] ^ lo8[1][bits[1]]
                          ^ lo8[2][bits[2]] ^ lo8[3][bits[3]])
                    ind = _mod_p(_mod_p(xh) * 256 + xl)
                w = w01[0][bits[0]] * w01[1][bits[1]]
                w = w * w01[2][bits[2]]
                w = w * w01[3][bits[3]]
                ind2 = ind + ind
                idxv[k * (NPH // CH) + row8, pl.ds(col, 16)] = ind2
                idxv[(FOFF // CH) + k * (NPH // CH) + row8,
                     pl.ds(col, 16)] = ind2 + 1
                wbuf[k, pl.ds(16 * g, 16)] = w
            return g + 1

        return grp

    NB = NPH // CH

    def fire_one(j):
        pltpu.async_copy(tbl.at[idxv.at[j]], rows.at[pl.ds(j * CH, CH)], sem)

    def drain(_, j):
        pltpu.make_async_copy(
            tbl.at[idxv.at[j]], rows.at[pl.ds(j * CH, CH)], sem).wait()
        return j + 1

    def combine_pass(l):
        def grp(_, g):
            n16 = 16 * g + iota
            acc0 = jnp.zeros((16,), jnp.float32)
            acc1 = jnp.zeros((16,), jnp.float32)
            for k in range(16):
                o = k * NPH + 16 * g
                f0 = rows[pl.ds(o, 16)]
                f1 = rows[pl.ds(FOFF + o, 16)]
                w = wbuf[k, pl.ds(16 * g, 16)]
                acc0 = acc0 + w * f0
                acc1 = acc1 + w * f1
            c0 = jnp.full((16,), 2 * l, jnp.int32)
            plsc.store_scatter(obuf, [n16, c0], acc0)
            plsc.store_scatter(obuf, [n16, c0 + 1], acc1)
            return g + 1

        lax.fori_loop(0, NGH, grp, np.int32(0))

    @pl.when(sid == 0)
    def _():
        pltpu.async_copy(datah.at[jnp.int32(0)], tbl, sem2)

    for h in range(2):
        p0 = h * NPH
        last = h == 1

        def run_level(l, is_direct, p0=p0, last=last):
            grp = make_grp(l, is_direct, p0)

            @pl.when(sid == 0)
            def _():
                pltpu.make_async_copy(datah.at[l], tbl, sem2).wait()
            plsc.subcore_barrier()

            def block(_, b):
                lax.fori_loop(0, 8, lambda i, g: grp(i, g), 8 * b)
                for k in range(16):
                    fire_one(k * NB + b)
                    fire_one(FOFF // CH + k * NB + b)
                return b + 1

            lax.fori_loop(0, NB, block, np.int32(0))
            lax.fori_loop(0, NCH2, drain, np.int32(0))
            plsc.subcore_barrier()

            cond = (sid == 0) & (l < np.int32(15)) if last else sid == 0

            @pl.when(cond)
            def _():
                nxt = jnp.bitwise_and(l + 1, np.int32(15))
                pltpu.async_copy(datah.at[nxt], tbl, sem2)

            combine_pass(l)

        run_level(jnp.int32(0), True)

        def hashed(_, l, run_level=run_level):
            run_level(l, False)
            return l + 1

        lax.fori_loop(1, NL, hashed, np.int32(1))
        pltpu.sync_copy(obuf, outh.at[pl.ds(base + p0, NPH)])


@jax.jit
def _run(coords, fco, ico, dataf):
    mesh = plsc.VectorSubcoreMesh(core_axis_name="c", subcore_axis_name="s")
    return pl.kernel(
        _sc_body,
        out_type=jax.ShapeDtypeStruct((N_PTS, NL * F), jnp.float32),
        mesh=mesh,
        compiler_params=pltpu.CompilerParams(
            needs_layout_passes=False, use_tc_tiling_on_sc=False),
        scratch_types=[
            pltpu.VMEM((4, NP), jnp.float32),
            pltpu.VMEM((4 * NL + 16,), jnp.float32),
            pltpu.VMEM((4 * NL + 16,), jnp.int32),
            pltpu.VMEM((NCH2, CH), jnp.int32),
            pltpu.VMEM((2 * 16 * NPH,), jnp.float32),
            pltpu.VMEM((16, NPH), jnp.float32),
            pltpu.VMEM((NPH, NL * F), jnp.float32),
            pltpu.VMEM_SHARED((_P * F,), jnp.float32),
            pltpu.SemaphoreType.DMA,
            pltpu.SemaphoreType.DMA,
        ],
    )(coords, fco, ico, dataf)


def kernel(xyz, t, data):
    coords = jnp.concatenate([xyz, t], axis=1).astype(jnp.float32).T
    fco = jnp.asarray(np.pad(_ENS.reshape(-1), (0, 16)))
    ico = jnp.asarray(np.pad(_ENN.reshape(-1) - 1, (0, 16)))
    dataf = data.astype(jnp.float32).reshape(NL, _P * F)
    return _run(coords, fco, ico, dataf)


# speedup vs baseline: 10.0779x; 1.0619x over previous
import jax
import jax.numpy as jnp
import numpy as np
from jax import lax
from jax.experimental import pallas as pl
from jax.experimental.pallas import tpu as pltpu
from jax.experimental.pallas import tpu_sc as plsc

N_PTS = 32768
NL = 16
F = 2


def _consts():
    nl = NL
    b = 1.38
    base = 16
    P = 2 ** 19
    def isprime(n):
        i = 2
        while i * i <= n:
            if n % i == 0:
                return False
            i += 1
        return True
    while not isprime(P):
        P += 1
    xyz_min = np.array([-1.0, -1.0, -1.0])
    xyz_max = np.array([1.0, 1.0, 1.0])
    ens, enn = [], []
    for i in range(nl):
        grid_num = int((base * b ** i) ** 3)
        grid_size = ((xyz_max - xyz_min).prod() / grid_num) ** (1.0 / 3.0)
        world_size = (xyz_max - xyz_min) / grid_size
        xyzt_num = np.concatenate([world_size, np.array([100])]).astype(np.int32)
        enn.append(xyzt_num)
        xyz_size = (xyz_max - xyz_min) / (world_size - 1)
        xyzt_size = np.concatenate([xyz_size, np.array([1.0 / 99])])
        ens.append(xyzt_size)
    return P, np.array(ens, dtype=np.float32), np.array(enn, dtype=np.int32)


_P, _ENS, _ENN = _consts()
_R = _P - (1 << 19)
_PS = (1, 19349663, 83492791, 73856093)

NW = 32
NP = N_PTS // NW
NPH = NP // 2
NGH = NPH // 16
CH = 128
NCH2 = 2 * 16 * NPH // CH
FOFF = 16 * NPH
PFP = ((_P * F + 127) // 128) * 128


def _mod_p(v):
    a = lax.shift_right_logical(v, np.int32(19))
    c = jnp.bitwise_and(v, np.int32((1 << 19) - 1))
    m = c - _R * a
    return jnp.where(m < 0, m + _P, m)


def _sc_body(coords, fcoh, icoh, datah, outh,
             cvm, fcv, icv, idxv, rows, wbuf, obuf, tbl, sem, sem2):
    sid = lax.axis_index("s")
    wid = sid * 2 + lax.axis_index("c")
    base = wid * NP
    pltpu.sync_copy(coords.at[:, pl.ds(base, NP)], cvm)
    pltpu.sync_copy(fcoh, fcv)
    pltpu.sync_copy(icoh, icv)
    iota = lax.broadcasted_iota(jnp.int32, (16,), 0)

    def make_grp(l, is_direct, p0):
        fvec = fcv[pl.ds(4 * l, 16)]
        ivec = icv[pl.ds(4 * l, 16)]
        es = [fvec[d] for d in range(4)]
        nm1 = [ivec[d] for d in range(4)]

        def grp(_, g):
            col = 16 * (g & np.int32(7))
            row8 = g >> np.int32(3)
            hi8 = []
            lo8 = []
            w01 = []
            for d in range(4):
                x = cvm[d, pl.ds(p0 + 16 * g, 16)]
                if d < 3:
                    fx = (x + np.float32(1.0)) / es[d]
                else:
                    fx = x / es[d]
                i0 = fx.astype(jnp.int32)
                i1 = (fx + np.float32(1.0)).astype(jnp.int32)
                frac = fx - i0.astype(jnp.float32)
                i0 = jnp.maximum(jnp.minimum(i0, nm1[d]), 0)
                i1 = jnp.maximum(jnp.minimum(i1, nm1[d]), 0)
                w01.append((np.float32(1.0) - frac, frac))
                if is_direct:
                    hi8.append((i0, i1))
                else:
                    chi = _PS[d] >> 8
                    clo = _PS[d] & 255
                    hs = []
                    ls = []
                    for v in (i0, i1):
                        tlo = v * clo
                        ls.append(jnp.bitwise_and(tlo, np.int32(255)))
                        hs.append(v * chi
                                  + lax.shift_right_logical(tlo, np.int32(8)))
                    hi8.append(tuple(hs))
                    lo8.append(tuple(ls))
            for k in range(16):
                bits = ((k >> 3) & 1, (k >> 2) & 1, (k >> 1) & 1, k & 1)
                if is_direct:
                    e1 = int(_ENN[0][1]); e2 = int(_ENN[0][2]); e3 = int(_ENN[0][3])
                    ind = (hi8[0][bits[0]] * (e1 * e2 * e3)
                           + hi8[1][bits[1]] * (e2 * e3)
                           + hi8[2][bits[2]] * e3
                           + hi8[3][bits[3]])
                else:
                    xh = (hi8[0][bits[0]] ^ hi8[1][bits[1]]
                          ^ hi8[2][bits[2]] ^ hi8[3][bits[3]])
                    xl = (lo8[0][bits[0]] ^ lo8[1][bits[1]]
                          ^ lo8[2][bits[2]] ^ lo8[3][bits[3]])
                    ind = _mod_p(_mod_p(xh) * 256 + xl)
                w = w01[0][bits[0]] * w01[1][bits[1]]
                w = w * w01[2][bits[2]]
                w = w * w01[3][bits[3]]
                ind2 = ind + ind
                idxv[k * (NPH // CH) + row8, pl.ds(col, 16)] = ind2
                idxv[(FOFF // CH) + k * (NPH // CH) + row8,
                     pl.ds(col, 16)] = ind2 + 1
                wbuf[k, pl.ds(16 * g, 16)] = w
            return g + 1

        return grp

    NB = NPH // CH

    def fire_one(j):
        pltpu.async_copy(tbl.at[idxv.at[j]], rows.at[pl.ds(j * CH, CH)], sem)

    def drain(_, j):
        pltpu.make_async_copy(
            tbl.at[idxv.at[j]], rows.at[pl.ds(j * CH, CH)], sem).wait()
        return j + 1

    def combine_pass(l):
        def grp(_, g):
            n16 = 16 * g + iota
            acc0 = jnp.zeros((16,), jnp.float32)
            acc1 = jnp.zeros((16,), jnp.float32)
            for k in range(16):
                o = k * NPH + 16 * g
                f0 = rows[pl.ds(o, 16)]
                f1 = rows[pl.ds(FOFF + o, 16)]
                w = wbuf[k, pl.ds(16 * g, 16)]
                acc0 = acc0 + w * f0
                acc1 = acc1 + w * f1
            c0 = jnp.full((16,), 2 * l, jnp.int32)
            plsc.store_scatter(obuf, [n16, c0], acc0)
            plsc.store_scatter(obuf, [n16, c0 + 1], acc1)
            return g + 1

        lax.fori_loop(0, NGH, grp, np.int32(0))

    SW = PFP // 16
    soff = sid * SW

    def stage_start(l):
        pltpu.async_copy(datah.at[l, pl.ds(soff, SW)],
                         tbl.at[pl.ds(soff, SW)], sem2)

    def stage_wait(l):
        pltpu.make_async_copy(datah.at[l, pl.ds(soff, SW)],
                              tbl.at[pl.ds(soff, SW)], sem2).wait()

    stage_start(jnp.int32(0))

    for h in range(2):
        p0 = h * NPH
        last = h == 1

        def run_level(l, is_direct, p0=p0, last=last):
            grp = make_grp(l, is_direct, p0)

            stage_wait(l)
            plsc.subcore_barrier()

            def block(_, b):
                lax.fori_loop(0, 8, lambda i, g: grp(i, g), 8 * b)
                for k in range(16):
                    fire_one(k * NB + b)
                    fire_one(FOFF // CH + k * NB + b)
                return b + 1

            lax.fori_loop(0, NB, block, np.int32(0))
            lax.fori_loop(0, NCH2, drain, np.int32(0))
            plsc.subcore_barrier()

            nxt = jnp.bitwise_and(l + 1, np.int32(15))
            if last:
                @pl.when(l < np.int32(15))
                def _():
                    stage_start(nxt)
            else:
                stage_start(nxt)

            combine_pass(l)

        run_level(jnp.int32(0), True)

        def hashed(_, l, run_level=run_level):
            run_level(l, False)
            return l + 1

        lax.fori_loop(1, NL, hashed, np.int32(1))
        pltpu.sync_copy(obuf, outh.at[pl.ds(base + p0, NPH)])


@jax.jit
def _run(coords, fco, ico, dataf):
    mesh = plsc.VectorSubcoreMesh(core_axis_name="c", subcore_axis_name="s")
    return pl.kernel(
        _sc_body,
        out_type=jax.ShapeDtypeStruct((N_PTS, NL * F), jnp.float32),
        mesh=mesh,
        compiler_params=pltpu.CompilerParams(
            needs_layout_passes=False, use_tc_tiling_on_sc=False),
        scratch_types=[
            pltpu.VMEM((4, NP), jnp.float32),
            pltpu.VMEM((4 * NL + 16,), jnp.float32),
            pltpu.VMEM((4 * NL + 16,), jnp.int32),
            pltpu.VMEM((NCH2, CH), jnp.int32),
            pltpu.VMEM((2 * 16 * NPH,), jnp.float32),
            pltpu.VMEM((16, NPH), jnp.float32),
            pltpu.VMEM((NPH, NL * F), jnp.float32),
            pltpu.VMEM_SHARED((PFP,), jnp.float32),
            pltpu.SemaphoreType.DMA,
            pltpu.SemaphoreType.DMA,
        ],
    )(coords, fco, ico, dataf)


def kernel(xyz, t, data):
    coords = jnp.concatenate([xyz, t], axis=1).astype(jnp.float32).T
    fco = jnp.asarray(np.pad(_ENS.reshape(-1), (0, 16)))
    ico = jnp.asarray(np.pad(_ENN.reshape(-1) - 1, (0, 16)))
    dataf = jnp.pad(data.astype(jnp.float32).reshape(NL, _P * F),
                    ((0, 0), (0, PFP - _P * F)))
    return _run(coords, fco, ico, dataf)
